# speedup vs baseline: 4.5519x; 4.5519x over previous
import jax
import jax.numpy as jnp
from jax.experimental import pallas as pl

L = 1024
KNN = 30
HI = jax.lax.Precision.HIGHEST


def _body(xf_ref, W1_ref, b1_ref, g1_ref, be1_ref, rm1_ref, rv1_ref,
          W2_ref, b2_ref, g2_ref, be2_ref, rm2_ref, rv2_ref, out_ref):
    ft = xf_ref[0]
    sq = jnp.sum(ft * ft, axis=1, keepdims=True)
    sq_row = jnp.reshape(jnp.sum(ft * ft, axis=1), (1, L))
    G = jax.lax.dot_general(ft, ft, (((1,), (1,)), ((), ())))
    d2 = sq + sq_row - 2.0 * G

    cols = jax.lax.broadcasted_iota(jnp.int32, (L, L), 1)

    def step(_, carry):
        vals, M = carry
        m = jnp.min(vals, axis=1, keepdims=True)
        eq = vals == m
        idxm = jnp.min(jnp.where(eq, cols, L), axis=1, keepdims=True)
        sel = cols == idxm
        M = M + sel.astype(jnp.float32)
        vals = jnp.where(sel, jnp.inf, vals)
        return vals, M

    _, M = jax.lax.fori_loop(
        0, KNN, step, (d2, jnp.zeros((L, L), jnp.float32)))

    ones_col = jnp.ones((L, 1), jnp.float32)
    deg = jax.lax.dot_general(M, ones_col, (((0,), (0,)), ((), ())),
                              precision=HI)
    degc = jnp.maximum(deg, 1.0)

    def bn(h, g_r, be_r, rm_r, rv_r):
        return (h - rm_r[0]) / jnp.sqrt(rv_r[0] + 1e-5) * g_r[0] + be_r[0]

    def v2v(h):
        E = jax.lax.dot_general(M, h, (((1,), (0,)), ((), ())),
                                precision=HI) * (1.0 / KNN)
        Vsum = jax.lax.dot_general(M, E, (((0,), (0,)), ((), ())),
                                   precision=HI)
        return Vsum / degc

    h = jax.lax.dot_general(ft, W1_ref[...], (((1,), (0,)), ((), ())))
    h = bn(h + b1_ref[0], g1_ref, be1_ref, rm1_ref, rv1_ref)
    h = jax.nn.relu(v2v(h))
    h = jax.lax.dot_general(h, W2_ref[...], (((1,), (0,)), ((), ())))
    h = bn(h + b2_ref[0], g2_ref, be2_ref, rm2_ref, rv2_ref)
    out_ref[0] = v2v(h)


def kernel(x, W1, b1, g1, be1, rm1, rv1, W2, b2, g2, be2, rm2, rv2):
    B, C, H, W = x.shape
    xf = x.reshape(B, L, C)
    vec = lambda v: v.reshape(1, -1)
    full = lambda r: pl.BlockSpec((1, r.shape[1]), lambda i: (0, 0))

    out = pl.pallas_call(
        _body,
        grid=(B,),
        in_specs=[
            pl.BlockSpec((1, L, C), lambda i: (i, 0, 0)),
            pl.BlockSpec(W1.shape, lambda i: (0, 0)),
            full(vec(b1)), full(vec(g1)), full(vec(be1)),
            full(vec(rm1)), full(vec(rv1)),
            pl.BlockSpec(W2.shape, lambda i: (0, 0)),
            full(vec(b2)), full(vec(g2)), full(vec(be2)),
            full(vec(rm2)), full(vec(rv2)),
        ],
        out_specs=pl.BlockSpec((1, L, W2.shape[1]), lambda i: (i, 0, 0)),
        out_shape=jax.ShapeDtypeStruct((B, L, W2.shape[1]), jnp.float32),
    )(xf, W1, vec(b1), vec(g1), vec(be1), vec(rm1), vec(rv1),
      W2, vec(b2), vec(g2), vec(be2), vec(rm2), vec(rv2))

    return out.reshape(B, -1, H, W)


# speedup vs baseline: 6.2032x; 1.3628x over previous
import jax
import jax.numpy as jnp
from jax.experimental import pallas as pl
from jax.experimental.pallas import tpu as pltpu

L = 1024
KNN = 30
HI = jax.lax.Precision.HIGHEST


def _body(xf_ref, W1_ref, b1_ref, g1_ref, be1_ref, rm1_ref, rv1_ref,
          W2_ref, b2_ref, g2_ref, be2_ref, rm2_ref, rv2_ref, out_ref,
          vals_ref):
    ft = xf_ref[0]
    sq = jnp.sum(ft * ft, axis=1, keepdims=True)
    sq_row = jnp.reshape(jnp.sum(ft * ft, axis=1), (1, L))
    G = jax.lax.dot_general(ft, ft, (((1,), (1,)), ((), ())))
    vals_ref[...] = sq + sq_row - 2.0 * G

    cols = jax.lax.broadcasted_iota(jnp.int32, (L, L), 1)
    tcols = jax.lax.broadcasted_iota(jnp.int32, (L, 32), 1)

    def step(t, nbr):
        vals = vals_ref[...]
        m = jnp.min(vals, axis=1, keepdims=True)
        eq = vals == m
        idxm = jnp.min(jnp.where(eq, cols, L), axis=1, keepdims=True)
        vals_ref[...] = jnp.where(cols == idxm, jnp.inf, vals)
        return jnp.where(tcols == t, idxm, nbr)

    nbr = jax.lax.fori_loop(
        0, KNN, step, jnp.zeros((L, 32), jnp.int32), unroll=2)

    M = jnp.zeros((L, L), jnp.float32)
    for t in range(KNN):
        M = M + (cols == nbr[:, t:t + 1]).astype(jnp.float32)

    ones_col = jnp.ones((L, 1), jnp.float32)
    deg = jax.lax.dot_general(M, ones_col, (((0,), (0,)), ((), ())),
                              precision=HI)
    degc = jnp.maximum(deg, 1.0)

    def bn(h, g_r, be_r, rm_r, rv_r):
        return (h - rm_r[0]) / jnp.sqrt(rv_r[0] + 1e-5) * g_r[0] + be_r[0]

    def v2v(h):
        E = jax.lax.dot_general(M, h, (((1,), (0,)), ((), ())),
                                precision=HI) * (1.0 / KNN)
        Vsum = jax.lax.dot_general(M, E, (((0,), (0,)), ((), ())),
                                   precision=HI)
        return Vsum / degc

    h = jax.lax.dot_general(ft, W1_ref[...], (((1,), (0,)), ((), ())))
    h = bn(h + b1_ref[0], g1_ref, be1_ref, rm1_ref, rv1_ref)
    h = jax.nn.relu(v2v(h))
    h = jax.lax.dot_general(h, W2_ref[...], (((1,), (0,)), ((), ())))
    h = bn(h + b2_ref[0], g2_ref, be2_ref, rm2_ref, rv2_ref)
    out_ref[0] = v2v(h)


def kernel(x, W1, b1, g1, be1, rm1, rv1, W2, b2, g2, be2, rm2, rv2):
    B, C, H, W = x.shape
    xf = x.reshape(B, L, C)
    vec = lambda v: v.reshape(1, -1)
    full = lambda r: pl.BlockSpec((1, r.shape[1]), lambda i: (0, 0))

    out = pl.pallas_call(
        _body,
        grid=(B,),
        in_specs=[
            pl.BlockSpec((1, L, C), lambda i: (i, 0, 0)),
            pl.BlockSpec(W1.shape, lambda i: (0, 0)),
            full(vec(b1)), full(vec(g1)), full(vec(be1)),
            full(vec(rm1)), full(vec(rv1)),
            pl.BlockSpec(W2.shape, lambda i: (0, 0)),
            full(vec(b2)), full(vec(g2)), full(vec(be2)),
            full(vec(rm2)), full(vec(rv2)),
        ],
        out_specs=pl.BlockSpec((1, L, W2.shape[1]), lambda i: (i, 0, 0)),
        out_shape=jax.ShapeDtypeStruct((B, L, W2.shape[1]), jnp.float32),
        scratch_shapes=[pltpu.VMEM((L, L), jnp.float32)],
    )(xf, W1, vec(b1), vec(g1), vec(be1), vec(rm1), vec(rv1),
      W2, vec(b2), vec(g2), vec(be2), vec(rm2), vec(rv2))

    return out.reshape(B, -1, H, W)


# speedup vs baseline: 8.5093x; 1.3718x over previous
import jax
import jax.numpy as jnp
from jax.experimental import pallas as pl
from jax.experimental.pallas import tpu as pltpu

L = 1024
KNN = 30
BF = jnp.bfloat16
F32 = jnp.float32


def _body(xf_ref, W1_ref, b1_ref, g1_ref, be1_ref, rm1_ref, rv1_ref,
          W2_ref, b2_ref, g2_ref, be2_ref, rm2_ref, rv2_ref, out_ref,
          vals_ref):
    ft = xf_ref[0]
    sq = jnp.sum(ft * ft, axis=1, keepdims=True)
    sq_row = jnp.reshape(jnp.sum(ft * ft, axis=1), (1, L))
    G = jax.lax.dot_general(ft, ft, (((1,), (1,)), ((), ())))
    vals_ref[...] = sq + sq_row - 2.0 * G

    cols = jax.lax.broadcasted_iota(jnp.int32, (L, L), 1)
    tcols = jax.lax.broadcasted_iota(jnp.int32, (L, 32), 1)

    def step(t, nbr):
        vals = vals_ref[...]
        m = jnp.min(vals, axis=1, keepdims=True)
        eq = vals == m
        idxm = jnp.min(jnp.where(eq, cols, L), axis=1, keepdims=True)
        vals_ref[...] = jnp.where(cols == idxm, jnp.inf, vals)
        return jnp.where(tcols == t, idxm, nbr)

    nbr = jax.lax.fori_loop(
        0, KNN, step, jnp.zeros((L, 32), jnp.int32), unroll=2)

    M = jnp.zeros((L, L), jnp.float32)
    for t in range(KNN):
        M = M + (cols == nbr[:, t:t + 1]).astype(jnp.float32)

    Mb = M.astype(BF)
    ones_col = jnp.ones((L, 1), BF)
    deg = jax.lax.dot_general(Mb, ones_col, (((0,), (0,)), ((), ())),
                              preferred_element_type=F32)
    degc = jnp.maximum(deg, 1.0)

    def bn(h, g_r, be_r, rm_r, rv_r):
        return (h - rm_r[0]) / jnp.sqrt(rv_r[0] + 1e-5) * g_r[0] + be_r[0]

    def mdot(h, dims):
        h_hi = h.astype(BF)
        h_lo = (h - h_hi.astype(F32)).astype(BF)
        return (jax.lax.dot_general(Mb, h_hi, dims, preferred_element_type=F32)
                + jax.lax.dot_general(Mb, h_lo, dims, preferred_element_type=F32))

    def v2v(h):
        E = mdot(h, (((1,), (0,)), ((), ()))) * (1.0 / KNN)
        Vsum = mdot(E, (((0,), (0,)), ((), ())))
        return Vsum / degc

    h = jax.lax.dot_general(ft, W1_ref[...], (((1,), (0,)), ((), ())))
    h = bn(h + b1_ref[0], g1_ref, be1_ref, rm1_ref, rv1_ref)
    h = jax.nn.relu(v2v(h))
    h = jax.lax.dot_general(h, W2_ref[...], (((1,), (0,)), ((), ())))
    h = bn(h + b2_ref[0], g2_ref, be2_ref, rm2_ref, rv2_ref)
    out_ref[0] = v2v(h)


def kernel(x, W1, b1, g1, be1, rm1, rv1, W2, b2, g2, be2, rm2, rv2):
    B, C, H, W = x.shape
    xf = x.reshape(B, L, C)
    vec = lambda v: v.reshape(1, -1)
    full = lambda r: pl.BlockSpec((1, r.shape[1]), lambda i: (0, 0))

    out = pl.pallas_call(
        _body,
        grid=(B,),
        in_specs=[
            pl.BlockSpec((1, L, C), lambda i: (i, 0, 0)),
            pl.BlockSpec(W1.shape, lambda i: (0, 0)),
            full(vec(b1)), full(vec(g1)), full(vec(be1)),
            full(vec(rm1)), full(vec(rv1)),
            pl.BlockSpec(W2.shape, lambda i: (0, 0)),
            full(vec(b2)), full(vec(g2)), full(vec(be2)),
            full(vec(rm2)), full(vec(rv2)),
        ],
        out_specs=pl.BlockSpec((1, L, W2.shape[1]), lambda i: (i, 0, 0)),
        out_shape=jax.ShapeDtypeStruct((B, L, W2.shape[1]), jnp.float32),
        scratch_shapes=[pltpu.VMEM((L, L), jnp.float32)],
    )(xf, W1, vec(b1), vec(g1), vec(be1), vec(rm1), vec(rv1),
      W2, vec(b2), vec(g2), vec(be2), vec(rm2), vec(rv2))

    return out.reshape(B, -1, H, W)
